# baseline (speedup 1.0000x reference)
import functools

import jax
import jax.numpy as jnp
from jax import lax
from jax.experimental import pallas as pl
from jax.experimental.pallas import tpu as pltpu
from jax.experimental.pallas import tpu_sc as plsc

N = 10000
NP = 10240
D = 256
H = 128
O = 70
E = 160000
EA = E + N

P_CH = 128
P_NCH = 42
P_PER_TILE = P_NCH * P_CH
P_TOT = 32 * P_PER_TILE
D_NCH = 86
D_PER_TILE = D_NCH * P_CH
D_TOT = 16 * D_PER_TILE

ROWS_PER_TILE = NP // 16
PROP_ROWS = 632
PROP_NP = 16 * PROP_ROWS

_mesh = plsc.VectorSubcoreMesh(core_axis_name="c", subcore_axis_name="s")
_sc_params = pltpu.CompilerParams(needs_layout_passes=False)

_f32 = jnp.float32
_i32 = jnp.int32


def _rsqrt_newton(x):
    i = lax.bitcast_convert_type(x, _i32)
    i = jnp.full((16,), 0x5F3759DF, _i32) - lax.shift_right_logical(i, 1)
    y = lax.bitcast_convert_type(i, _f32)
    for _ in range(3):
        y = y * (1.5 - 0.5 * x * y * y)
    return y


def _deg_body(dst_c, ew_f, zdeg, dinv_out, dstv, ewv, degl, dinvl, acc):
    c = lax.axis_index("c")
    s = lax.axis_index("s")
    pltpu.sync_copy(dst_c.at[s], dstv)
    pltpu.sync_copy(ew_f.at[s], ewv)
    pltpu.sync_copy(zdeg, acc.at[pl.ds(s * ROWS_PER_TILE, ROWS_PER_TILE)])
    plsc.subcore_barrier()

    def chunk(j, _):
        pltpu.sync_copy(ewv.at[pl.ds(j * P_CH, P_CH)], acc.at[dstv.at[j]],
                        add=True)
        return 0

    lax.fori_loop(0, D_NCH, chunk, 0)
    plsc.subcore_barrier()

    base = (c * 16 + s) * (ROWS_PER_TILE // 2)
    pltpu.sync_copy(acc.at[pl.ds(base, ROWS_PER_TILE // 2)], degl)

    def grp(g, _):
        x = degl[pl.ds(g * 16, 16)]
        dinvl[pl.ds(g * 16, 16)] = _rsqrt_newton(x)
        return 0

    lax.fori_loop(0, ROWS_PER_TILE // 32, grp, 0)
    pltpu.sync_copy(dinvl, dinv_out.at[pl.ds(base, ROWS_PER_TILE // 2)])


_deg_kernel = pl.kernel(
    _deg_body,
    out_type=jax.ShapeDtypeStruct((NP,), _f32),
    mesh=_mesh,
    compiler_params=_sc_params,
    scratch_types=[
        pltpu.VMEM((D_NCH, P_CH), _i32),
        pltpu.VMEM((D_PER_TILE,), _f32),
        pltpu.VMEM((ROWS_PER_TILE // 2,), _f32),
        pltpu.VMEM((ROWS_PER_TILE // 2,), _f32),
        pltpu.VMEM_SHARED((NP,), _f32),
    ],
)


NBUF = 2
GSPLIT = 2
HH = H // 2
GR = P_CH // GSPLIT


def _prop_body(h_t, src_f, dst_c, ew_f, parts_out,
               srcv, dstv, ewv, rows, gsems, ssems, isems, wsems, acc):
    c = lax.axis_index("c")
    s = lax.axis_index("s")
    w = 2 * s + c
    base_row = s * PROP_ROWS
    ic0 = pltpu.async_copy(src_f.at[w], srcv, isems[0])
    ic1 = pltpu.async_copy(dst_c.at[w], dstv, isems[1])
    ic2 = pltpu.async_copy(ew_f.at[w], ewv, isems[2])
    zv = jnp.zeros((16,), _f32)

    def zrow(i, _):
        for zc in range(H // 16):
            rows[0][i, pl.ds(zc * 16, 16)] = zv
        return 0

    lax.fori_loop(0, P_CH, zrow, 0)
    zcps = []
    for q, (off, ln) in enumerate(
            ((0, 128), (128, 128), (256, 128), (384, 128), (512, 120))):
        sem = wsems[q] if q < 4 else gsems[0]
        zcps.append(pltpu.async_copy(
            rows[0].at[pl.ds(0, ln)],
            acc.at[pl.ds(base_row + off, ln)], sem))
    for cp in zcps:
        cp.wait()
    ic0.wait()
    ic1.wait()
    ic2.wait()
    plsc.subcore_barrier()

    def group(t, _):
        j0 = t * NBUF
        gets = []
        for b in range(NBUF):
            for g in range(GSPLIT):
                idx = srcv.at[pl.ds((j0 + b) * P_CH + g * GR, GR)]
                gets.append(pltpu.async_copy(
                    h_t.at[idx], rows[b].at[pl.ds(g * GR, GR)],
                    gsems[b * GSPLIT + g]))
        puts = []
        for b in range(NBUF):
            for g in range(GSPLIT):
                gets[b * GSPLIT + g].wait()

            def rbody(r, _, _b=b):
                f = plsc.load_gather(
                    ewv, [jnp.full((16,), (j0 + _b) * P_CH + r, _i32)])
                for cc in range(H // 16):
                    sl = pl.ds(cc * 16, 16)
                    rows[_b][r, sl] = rows[_b][r, sl] * f
                return 0

            lax.fori_loop(0, P_CH, rbody, 0)
            puts.append(pltpu.async_copy(rows[b], acc.at[dstv.at[j0 + b]],
                                         ssems[b], add=True))
        for cp in puts:
            cp.wait()
        return 0

    lax.fori_loop(0, P_NCH // NBUF, group, 0)
    plsc.subcore_barrier()
    wcps = []
    for q, (off, ln) in enumerate(((0, 160), (160, 160), (320, 160), (480, 152))):
        sl = pl.ds(base_row + off, ln)
        wcps.append(pltpu.async_copy(acc.at[sl], parts_out.at[c].at[sl],
                                     wsems[q]))
    for cp in wcps:
        cp.wait()


_prop_kernel = pl.kernel(
    _prop_body,
    out_type=jax.ShapeDtypeStruct((2, PROP_NP, H), _f32),
    mesh=_mesh,
    compiler_params=_sc_params,
    scratch_types=[
        pltpu.VMEM((P_PER_TILE,), _i32),
        pltpu.VMEM((P_NCH, P_CH), _i32),
        pltpu.VMEM((P_PER_TILE,), _f32),
        [pltpu.VMEM((P_CH, H), _f32) for _ in range(NBUF)],
        [pltpu.SemaphoreType.DMA for _ in range(NBUF * GSPLIT)],
        [pltpu.SemaphoreType.DMA for _ in range(NBUF)],
        [pltpu.SemaphoreType.DMA for _ in range(3)],
        [pltpu.SemaphoreType.DMA for _ in range(4)],
        pltpu.VMEM_SHARED((PROP_NP, H), _f32),
    ],
)


BR = 1000


def _k1_body(x_ref, w1_ref, dinv_ref, out_ref):
    h = lax.dot_general(x_ref[...], w1_ref[...], (((1,), (0,)), ((), ())),
                        preferred_element_type=_f32)
    out_ref[...] = h * dinv_ref[...]


def _tc_k1(x, W1, dinv2d):
    return pl.pallas_call(
        _k1_body,
        grid=(N // BR,),
        in_specs=[
            pl.BlockSpec((BR, D), lambda i: (i, 0)),
            pl.BlockSpec((D, H), lambda i: (0, 0)),
            pl.BlockSpec((BR, 1), lambda i: (i, 0)),
        ],
        out_specs=pl.BlockSpec((BR, H), lambda i: (i, 0)),
        out_shape=jax.ShapeDtypeStruct((N, H), _f32),
    )(x, W1, dinv2d)


def _k2_body(parts_ref, dinv_ref, b1_ref, z_ref, sums_ref):
    i = pl.program_id(0)
    z = dinv_ref[...] * (parts_ref[0] + parts_ref[1]) + b1_ref[...]
    z_ref[...] = z

    @pl.when(i == 0)
    def _():
        sums_ref[...] = jnp.zeros_like(sums_ref)

    sums_ref[0:1, :] += jnp.sum(z, axis=0, keepdims=True)
    sums_ref[1:2, :] += jnp.sum(z * z, axis=0, keepdims=True)


def _tc_k2(parts, dinv2d, b1):
    return pl.pallas_call(
        _k2_body,
        grid=(N // BR,),
        in_specs=[
            pl.BlockSpec((2, BR, H), lambda i: (0, i, 0)),
            pl.BlockSpec((BR, 1), lambda i: (i, 0)),
            pl.BlockSpec((H,), lambda i: (0,)),
        ],
        out_specs=[
            pl.BlockSpec((BR, H), lambda i: (i, 0)),
            pl.BlockSpec((2, H), lambda i: (0, 0)),
        ],
        out_shape=[
            jax.ShapeDtypeStruct((N, H), _f32),
            jax.ShapeDtypeStruct((2, H), _f32),
        ],
    )(parts, dinv2d, b1)


def _k3_body(z_ref, sums_ref, g1_ref, be1_ref, w2_ref, dinv_ref, out_ref):
    inv_n = 1.0 / N
    m = sums_ref[0:1, :] * inv_n
    var = sums_ref[1:2, :] * inv_n - m * m
    scale = g1_ref[...] * lax.rsqrt(var + 1e-5)
    h = jnp.maximum((z_ref[...] - m) * scale + be1_ref[...], 0.0)
    h2 = lax.dot_general(h, w2_ref[...], (((1,), (0,)), ((), ())),
                         preferred_element_type=_f32)
    out_ref[...] = h2 * dinv_ref[...]


def _tc_k3(z, sums, g1, be1, W2, dinv2d):
    return pl.pallas_call(
        _k3_body,
        grid=(N // BR,),
        in_specs=[
            pl.BlockSpec((BR, H), lambda i: (i, 0)),
            pl.BlockSpec((2, H), lambda i: (0, 0)),
            pl.BlockSpec((H,), lambda i: (0,)),
            pl.BlockSpec((H,), lambda i: (0,)),
            pl.BlockSpec((H, H), lambda i: (0, 0)),
            pl.BlockSpec((BR, 1), lambda i: (i, 0)),
        ],
        out_specs=pl.BlockSpec((BR, H), lambda i: (i, 0)),
        out_shape=jax.ShapeDtypeStruct((N, H), _f32),
    )(z, sums, g1, be1, W2, dinv2d)


def _k4_body(parts_ref, dinv_ref, b2_ref, wc_ref, bc_ref, out_ref):
    z2 = dinv_ref[...] * (parts_ref[0] + parts_ref[1]) + b2_ref[...]
    out = lax.dot_general(z2, wc_ref[...], (((1,), (0,)), ((), ())),
                          preferred_element_type=_f32)
    out_ref[...] = out + bc_ref[...]


def _tc_k4(parts, dinv2d, b2, Wc, bc):
    return pl.pallas_call(
        _k4_body,
        grid=(N // BR,),
        in_specs=[
            pl.BlockSpec((2, BR, H), lambda i: (0, i, 0)),
            pl.BlockSpec((BR, 1), lambda i: (i, 0)),
            pl.BlockSpec((H,), lambda i: (0,)),
            pl.BlockSpec((H, O), lambda i: (0, 0)),
            pl.BlockSpec((O,), lambda i: (0,)),
        ],
        out_specs=pl.BlockSpec((BR, O), lambda i: (i, 0)),
        out_shape=jax.ShapeDtypeStruct((N, O), _f32),
    )(parts, dinv2d, b2, Wc, bc)


def kernel(x, edge_index, edge_weight, W1, b1, g1, be1, W2, b2, g2, be2, Wc, bc):
    src = edge_index[0].astype(_i32)
    dst = edge_index[1].astype(_i32)
    loop = jnp.arange(N, dtype=_i32)
    src_a = jnp.concatenate([src, loop])
    dst_a = jnp.concatenate([dst, loop])
    ew_a = jnp.concatenate([edge_weight.astype(_f32), jnp.ones((N,), _f32)])

    pp = P_TOT - EA
    pad_p = jnp.arange(pp, dtype=_i32) % N
    src_p = jnp.concatenate([src_a, jnp.zeros((pp,), _i32)]).reshape(32, P_PER_TILE)
    dst_p = jnp.concatenate([dst_a, pad_p])
    dst_pc = dst_p.reshape(32, P_NCH, P_CH)
    ew_p = jnp.concatenate([ew_a, jnp.zeros((pp,), _f32)]).reshape(32, P_PER_TILE)

    dp = D_TOT - EA
    pad_d = jnp.arange(dp, dtype=_i32) % N
    dst_dc = jnp.concatenate([dst_a, pad_d]).reshape(16, D_NCH, P_CH)
    ew_d = jnp.concatenate([ew_a, jnp.zeros((dp,), _f32)]).reshape(16, D_PER_TILE)

    zdeg = jnp.zeros((ROWS_PER_TILE,), _f32)

    dinv = _deg_kernel(dst_dc, ew_d, zdeg)
    dinv2d = dinv[:N].reshape(N, 1)

    h1p = _tc_k1(x, W1, dinv2d)
    parts1 = _prop_kernel(h1p, src_p, dst_pc, ew_p)
    z, sums = _tc_k2(parts1, dinv2d, b1)
    h2p = _tc_k3(z, sums, g1, be1, W2, dinv2d)
    parts2 = _prop_kernel(h2p, src_p, dst_pc, ew_p)
    return _tc_k4(parts2, dinv2d, b2, Wc, bc)


# speedup vs baseline: 1.1895x; 1.1895x over previous
import functools

import jax
import jax.numpy as jnp
from jax import lax
from jax.experimental import pallas as pl
from jax.experimental.pallas import tpu as pltpu
from jax.experimental.pallas import tpu_sc as plsc

N = 10000
NP = 10240
D = 256
H = 128
O = 70
E = 160000
EA = E + N

P_CH = 128
P_WCH = 28
P_NCH = 42
P_PER_TILE = P_NCH * P_CH
P_TOT = 32 * P_PER_TILE
P_NCHUNKS = P_TOT // P_CH
C0_CH = 2 * P_WCH
C1_CH = P_WCH
D_NCH = 86
D_PER_TILE = D_NCH * P_CH
D_TOT = 16 * D_PER_TILE

ROWS_PER_TILE = NP // 16
PROP_ROWS = 632
PROP_NP = 16 * PROP_ROWS

_mesh = plsc.VectorSubcoreMesh(core_axis_name="c", subcore_axis_name="s")
_sc_params = pltpu.CompilerParams(needs_layout_passes=False)

_f32 = jnp.float32
_i32 = jnp.int32


def _rsqrt_newton(x):
    i = lax.bitcast_convert_type(x, _i32)
    i = jnp.full((16,), 0x5F3759DF, _i32) - lax.shift_right_logical(i, 1)
    y = lax.bitcast_convert_type(i, _f32)
    for _ in range(3):
        y = y * (1.5 - 0.5 * x * y * y)
    return y


def _deg_body(dst_c, ew_f, zdeg, dinv_out, dstv, ewv, degl, dinvl, acc):
    c = lax.axis_index("c")
    s = lax.axis_index("s")
    pltpu.sync_copy(dst_c.at[s], dstv)
    pltpu.sync_copy(ew_f.at[s], ewv)
    pltpu.sync_copy(zdeg, acc.at[pl.ds(s * ROWS_PER_TILE, ROWS_PER_TILE)])
    plsc.subcore_barrier()

    def chunk(j, _):
        pltpu.sync_copy(ewv.at[pl.ds(j * P_CH, P_CH)], acc.at[dstv.at[j]],
                        add=True)
        return 0

    lax.fori_loop(0, D_NCH, chunk, 0)
    plsc.subcore_barrier()

    base = (c * 16 + s) * (ROWS_PER_TILE // 2)
    pltpu.sync_copy(acc.at[pl.ds(base, ROWS_PER_TILE // 2)], degl)

    def grp(g, _):
        x = degl[pl.ds(g * 16, 16)]
        dinvl[pl.ds(g * 16, 16)] = _rsqrt_newton(x)
        return 0

    lax.fori_loop(0, ROWS_PER_TILE // 32, grp, 0)
    pltpu.sync_copy(dinvl, dinv_out.at[pl.ds(base, ROWS_PER_TILE // 2)])


_deg_kernel = pl.kernel(
    _deg_body,
    out_type=jax.ShapeDtypeStruct((NP,), _f32),
    mesh=_mesh,
    compiler_params=_sc_params,
    scratch_types=[
        pltpu.VMEM((D_NCH, P_CH), _i32),
        pltpu.VMEM((D_PER_TILE,), _f32),
        pltpu.VMEM((ROWS_PER_TILE // 2,), _f32),
        pltpu.VMEM((ROWS_PER_TILE // 2,), _f32),
        pltpu.VMEM_SHARED((NP,), _f32),
    ],
)


NBUF = 2


def _prop_body(h_t, src_f, dst_c, ew_f, parts_out,
               srcv, dstv, ewv, rows, gsems, ssems, isems, wsems, acc):
    c = lax.axis_index("c")
    s = lax.axis_index("s")
    base_row = s * PROP_ROWS
    zv = jnp.zeros((16,), _f32)

    def zrow(i, _):
        for zc in range(H // 16):
            rows[0][i, pl.ds(zc * 16, 16)] = zv
        return 0

    lax.fori_loop(0, P_CH, zrow, 0)
    zcps = []
    for q, (off, ln) in enumerate(
            ((0, 128), (128, 128), (256, 128), (384, 128), (512, 120))):
        sem = wsems[q] if q < 4 else gsems[0]
        zcps.append(pltpu.async_copy(
            rows[0].at[pl.ds(0, ln)],
            acc.at[pl.ds(base_row + off, ln)], sem))
    for cp in zcps:
        cp.wait()
    plsc.subcore_barrier()

    def do_wave(base_ch):
        ic0 = pltpu.async_copy(src_f.at[pl.ds(base_ch * P_CH, P_WCH * P_CH)],
                               srcv, isems[0])
        ic1 = pltpu.async_copy(dst_c.at[pl.ds(base_ch * P_CH, P_WCH * P_CH)],
                               dstv, isems[1])
        ic2 = pltpu.async_copy(ew_f.at[pl.ds(base_ch * P_CH, P_WCH * P_CH)],
                               ewv, isems[2])
        ic0.wait()
        ic1.wait()
        ic2.wait()

        def group(t, _):
            j0 = t * NBUF
            gets = []
            for b in range(NBUF):
                idx = srcv.at[pl.ds((j0 + b) * P_CH, P_CH)]
                gets.append(pltpu.async_copy(h_t.at[idx], rows[b], gsems[b]))
            puts = []
            for b in range(NBUF):
                gets[b].wait()

                def rbody(r, _, _b=b):
                    f = plsc.load_gather(
                        ewv, [jnp.full((16,), (j0 + _b) * P_CH + r, _i32)])
                    for cc in range(H // 16):
                        sl = pl.ds(cc * 16, 16)
                        rows[_b][r, sl] = rows[_b][r, sl] * f
                    return 0

                lax.fori_loop(0, P_CH, rbody, 0)
                didx = dstv.at[pl.ds((j0 + b) * P_CH, P_CH)]
                puts.append(pltpu.async_copy(rows[b], acc.at[didx],
                                             ssems[b], add=True))
            for cp in puts:
                cp.wait()
            return 0

        lax.fori_loop(0, P_WCH // NBUF, group, 0)

    @pl.when(c == 0)
    def _():
        do_wave(s * C0_CH)
        do_wave(s * C0_CH + P_WCH)

    @pl.when(c == 1)
    def _():
        do_wave(16 * C0_CH + s * C1_CH)

    plsc.subcore_barrier()
    wcps = []
    for q, (off, ln) in enumerate(((0, 160), (160, 160), (320, 160), (480, 152))):
        sl = pl.ds(base_row + off, ln)
        wcps.append(pltpu.async_copy(acc.at[sl], parts_out.at[c].at[sl],
                                     wsems[q]))
    for cp in wcps:
        cp.wait()


_prop_kernel = pl.kernel(
    _prop_body,
    out_type=jax.ShapeDtypeStruct((2, PROP_NP, H), _f32),
    mesh=_mesh,
    compiler_params=_sc_params,
    scratch_types=[
        pltpu.VMEM((P_WCH * P_CH,), _i32),
        pltpu.VMEM((P_WCH * P_CH,), _i32),
        pltpu.VMEM((P_WCH * P_CH,), _f32),
        [pltpu.VMEM((P_CH, H), _f32) for _ in range(NBUF)],
        [pltpu.SemaphoreType.DMA for _ in range(NBUF)],
        [pltpu.SemaphoreType.DMA for _ in range(NBUF)],
        [pltpu.SemaphoreType.DMA for _ in range(3)],
        [pltpu.SemaphoreType.DMA for _ in range(4)],
        pltpu.VMEM_SHARED((PROP_NP, H), _f32),
    ],
)


BR = 1000


def _k1_body(x_ref, w1_ref, dinv_ref, out_ref):
    h = lax.dot_general(x_ref[...], w1_ref[...], (((1,), (0,)), ((), ())),
                        preferred_element_type=_f32)
    out_ref[...] = h * dinv_ref[...]


def _tc_k1(x, W1, dinv2d):
    return pl.pallas_call(
        _k1_body,
        grid=(N // BR,),
        in_specs=[
            pl.BlockSpec((BR, D), lambda i: (i, 0)),
            pl.BlockSpec((D, H), lambda i: (0, 0)),
            pl.BlockSpec((BR, 1), lambda i: (i, 0)),
        ],
        out_specs=pl.BlockSpec((BR, H), lambda i: (i, 0)),
        out_shape=jax.ShapeDtypeStruct((N, H), _f32),
    )(x, W1, dinv2d)


def _k2_body(parts_ref, dinv_ref, b1_ref, z_ref, sums_ref):
    i = pl.program_id(0)
    z = dinv_ref[...] * (parts_ref[0] + parts_ref[1]) + b1_ref[...]
    z_ref[...] = z

    @pl.when(i == 0)
    def _():
        sums_ref[...] = jnp.zeros_like(sums_ref)

    sums_ref[0:1, :] += jnp.sum(z, axis=0, keepdims=True)
    sums_ref[1:2, :] += jnp.sum(z * z, axis=0, keepdims=True)


def _tc_k2(parts, dinv2d, b1):
    return pl.pallas_call(
        _k2_body,
        grid=(N // BR,),
        in_specs=[
            pl.BlockSpec((2, BR, H), lambda i: (0, i, 0)),
            pl.BlockSpec((BR, 1), lambda i: (i, 0)),
            pl.BlockSpec((H,), lambda i: (0,)),
        ],
        out_specs=[
            pl.BlockSpec((BR, H), lambda i: (i, 0)),
            pl.BlockSpec((2, H), lambda i: (0, 0)),
        ],
        out_shape=[
            jax.ShapeDtypeStruct((N, H), _f32),
            jax.ShapeDtypeStruct((2, H), _f32),
        ],
    )(parts, dinv2d, b1)


def _k3_body(z_ref, sums_ref, g1_ref, be1_ref, w2_ref, dinv_ref, out_ref):
    inv_n = 1.0 / N
    m = sums_ref[0:1, :] * inv_n
    var = sums_ref[1:2, :] * inv_n - m * m
    scale = g1_ref[...] * lax.rsqrt(var + 1e-5)
    h = jnp.maximum((z_ref[...] - m) * scale + be1_ref[...], 0.0)
    h2 = lax.dot_general(h, w2_ref[...], (((1,), (0,)), ((), ())),
                         preferred_element_type=_f32)
    out_ref[...] = h2 * dinv_ref[...]


def _tc_k3(z, sums, g1, be1, W2, dinv2d):
    return pl.pallas_call(
        _k3_body,
        grid=(N // BR,),
        in_specs=[
            pl.BlockSpec((BR, H), lambda i: (i, 0)),
            pl.BlockSpec((2, H), lambda i: (0, 0)),
            pl.BlockSpec((H,), lambda i: (0,)),
            pl.BlockSpec((H,), lambda i: (0,)),
            pl.BlockSpec((H, H), lambda i: (0, 0)),
            pl.BlockSpec((BR, 1), lambda i: (i, 0)),
        ],
        out_specs=pl.BlockSpec((BR, H), lambda i: (i, 0)),
        out_shape=jax.ShapeDtypeStruct((N, H), _f32),
    )(z, sums, g1, be1, W2, dinv2d)


def _k4_body(parts_ref, dinv_ref, b2_ref, wc_ref, bc_ref, out_ref):
    z2 = dinv_ref[...] * (parts_ref[0] + parts_ref[1]) + b2_ref[...]
    out = lax.dot_general(z2, wc_ref[...], (((1,), (0,)), ((), ())),
                          preferred_element_type=_f32)
    out_ref[...] = out + bc_ref[...]


def _tc_k4(parts, dinv2d, b2, Wc, bc):
    return pl.pallas_call(
        _k4_body,
        grid=(N // BR,),
        in_specs=[
            pl.BlockSpec((2, BR, H), lambda i: (0, i, 0)),
            pl.BlockSpec((BR, 1), lambda i: (i, 0)),
            pl.BlockSpec((H,), lambda i: (0,)),
            pl.BlockSpec((H, O), lambda i: (0, 0)),
            pl.BlockSpec((O,), lambda i: (0,)),
        ],
        out_specs=pl.BlockSpec((BR, O), lambda i: (i, 0)),
        out_shape=jax.ShapeDtypeStruct((N, O), _f32),
    )(parts, dinv2d, b2, Wc, bc)


def kernel(x, edge_index, edge_weight, W1, b1, g1, be1, W2, b2, g2, be2, Wc, bc):
    src = edge_index[0].astype(_i32)
    dst = edge_index[1].astype(_i32)
    loop = jnp.arange(N, dtype=_i32)
    src_a = jnp.concatenate([src, loop])
    dst_a = jnp.concatenate([dst, loop])
    ew_a = jnp.concatenate([edge_weight.astype(_f32), jnp.ones((N,), _f32)])

    pp = P_TOT - EA
    pad_p = jnp.arange(pp, dtype=_i32) % N
    src_p = jnp.concatenate([src_a, jnp.zeros((pp,), _i32)])
    dst_pc = jnp.concatenate([dst_a, pad_p])
    ew_p = jnp.concatenate([ew_a, jnp.zeros((pp,), _f32)])

    dp = D_TOT - EA
    pad_d = jnp.arange(dp, dtype=_i32) % N
    dst_dc = jnp.concatenate([dst_a, pad_d]).reshape(16, D_NCH, P_CH)
    ew_d = jnp.concatenate([ew_a, jnp.zeros((dp,), _f32)]).reshape(16, D_PER_TILE)

    zdeg = jnp.zeros((ROWS_PER_TILE,), _f32)

    dinv = _deg_kernel(dst_dc, ew_d, zdeg)
    dinv2d = dinv[:N].reshape(N, 1)

    h1p = _tc_k1(x, W1, dinv2d)
    parts1 = _prop_kernel(h1p, src_p, dst_pc, ew_p)
    z, sums = _tc_k2(parts1, dinv2d, b1)
    h2p = _tc_k3(z, sums, g1, be1, W2, dinv2d)
    parts2 = _prop_kernel(h2p, src_p, dst_pc, ew_p)
    return _tc_k4(parts2, dinv2d, b2, Wc, bc)


# speedup vs baseline: 1.1979x; 1.0071x over previous
import functools

import jax
import jax.numpy as jnp
from jax import lax
from jax.experimental import pallas as pl
from jax.experimental.pallas import tpu as pltpu
from jax.experimental.pallas import tpu_sc as plsc

N = 10000
NP = 10240
D = 256
H = 128
O = 70
E = 160000
EA = E + N

P_CH = 128
P_WCH = 28
P_NCH = 42
P_PER_TILE = P_NCH * P_CH
P_TOT = 32 * P_PER_TILE
P_NCHUNKS = P_TOT // P_CH
C0_CH = 2 * P_WCH
C1_CH = P_WCH
D_NCH = 86
D_PER_TILE = D_NCH * P_CH
D_TOT = 16 * D_PER_TILE

ROWS_PER_TILE = NP // 16
PROP_ROWS = 632
PROP_NP = 16 * PROP_ROWS

_mesh = plsc.VectorSubcoreMesh(core_axis_name="c", subcore_axis_name="s")
_sc_params = pltpu.CompilerParams(needs_layout_passes=False)

_f32 = jnp.float32
_i32 = jnp.int32


def _rsqrt_newton(x):
    i = lax.bitcast_convert_type(x, _i32)
    i = jnp.full((16,), 0x5F3759DF, _i32) - lax.shift_right_logical(i, 1)
    y = lax.bitcast_convert_type(i, _f32)
    for _ in range(3):
        y = y * (1.5 - 0.5 * x * y * y)
    return y


def _deg_body(dst_c, ew_f, zdeg, dinv_out, dstv, ewv, degl, dinvl, acc):
    c = lax.axis_index("c")
    s = lax.axis_index("s")
    pltpu.sync_copy(dst_c.at[s], dstv)
    pltpu.sync_copy(ew_f.at[s], ewv)
    pltpu.sync_copy(zdeg, acc.at[pl.ds(s * ROWS_PER_TILE, ROWS_PER_TILE)])
    plsc.subcore_barrier()

    def chunk(j, _):
        pltpu.sync_copy(ewv.at[pl.ds(j * P_CH, P_CH)], acc.at[dstv.at[j]],
                        add=True)
        return 0

    lax.fori_loop(0, D_NCH, chunk, 0)
    plsc.subcore_barrier()

    base = (c * 16 + s) * (ROWS_PER_TILE // 2)
    pltpu.sync_copy(acc.at[pl.ds(base, ROWS_PER_TILE // 2)], degl)

    def grp(g, _):
        x = degl[pl.ds(g * 16, 16)]
        dinvl[pl.ds(g * 16, 16)] = _rsqrt_newton(x)
        return 0

    lax.fori_loop(0, ROWS_PER_TILE // 32, grp, 0)
    pltpu.sync_copy(dinvl, dinv_out.at[pl.ds(base, ROWS_PER_TILE // 2)])


_deg_kernel = pl.kernel(
    _deg_body,
    out_type=jax.ShapeDtypeStruct((NP,), _f32),
    mesh=_mesh,
    compiler_params=_sc_params,
    scratch_types=[
        pltpu.VMEM((D_NCH, P_CH), _i32),
        pltpu.VMEM((D_PER_TILE,), _f32),
        pltpu.VMEM((ROWS_PER_TILE // 2,), _f32),
        pltpu.VMEM((ROWS_PER_TILE // 2,), _f32),
        pltpu.VMEM_SHARED((NP,), _f32),
    ],
)


NBUF = 2


def _prop_body(h_t, src_f, dst_c, ew_f, parts_out,
               srcv, dstv, ewv, rows, gsems, ssems, isems, wsems, acc):
    c = lax.axis_index("c")
    s = lax.axis_index("s")
    base_row = s * PROP_ROWS
    zv = jnp.zeros((16,), _f32)

    def zrow(i, _):
        for zc in range(H // 16):
            rows[0][i, pl.ds(zc * 16, 16)] = zv
        return 0

    lax.fori_loop(0, P_CH, zrow, 0)
    zcps = []
    for q, (off, ln) in enumerate(
            ((0, 128), (128, 128), (256, 128), (384, 128), (512, 120))):
        sem = wsems[q] if q < 4 else gsems[0]
        zcps.append(pltpu.async_copy(
            rows[0].at[pl.ds(0, ln)],
            acc.at[pl.ds(base_row + off, ln)], sem))
    for cp in zcps:
        cp.wait()
    plsc.subcore_barrier()

    def do_wave(base_ch):
        ic0 = pltpu.async_copy(src_f.at[pl.ds(base_ch * P_CH, P_WCH * P_CH)],
                               srcv, isems[0])
        ic1 = pltpu.async_copy(dst_c.at[pl.ds(base_ch * P_CH, P_WCH * P_CH)],
                               dstv, isems[1])
        ic2 = pltpu.async_copy(ew_f.at[pl.ds(base_ch * P_CH, P_WCH * P_CH)],
                               ewv, isems[2])
        ic0.wait()
        ic1.wait()
        ic2.wait()

        def group(t, _):
            j0 = t * NBUF
            gets = []
            for b in range(NBUF):
                idx = srcv.at[pl.ds((j0 + b) * P_CH, P_CH)]
                gets.append(pltpu.async_copy(h_t.at[idx], rows[b], gsems[b]))
            puts = []
            for b in range(NBUF):
                gets[b].wait()

                def rbody(r2, _, _b=b):
                    r = r2 * 2
                    for rr in range(2):
                        f = plsc.load_gather(
                            ewv,
                            [jnp.full((16,), (j0 + _b) * P_CH + r + rr,
                                      _i32)])
                        for cc in range(H // 16):
                            sl = pl.ds(cc * 16, 16)
                            rows[_b][r + rr, sl] = rows[_b][r + rr, sl] * f
                    return 0

                lax.fori_loop(0, P_CH // 2, rbody, 0)
                didx = dstv.at[pl.ds((j0 + b) * P_CH, P_CH)]
                puts.append(pltpu.async_copy(rows[b], acc.at[didx],
                                             ssems[b], add=True))
            for cp in puts:
                cp.wait()
            return 0

        lax.fori_loop(0, P_WCH // NBUF, group, 0)

    @pl.when(c == 0)
    def _():
        do_wave(s * C0_CH)
        do_wave(s * C0_CH + P_WCH)

    @pl.when(c == 1)
    def _():
        do_wave(16 * C0_CH + s * C1_CH)

    plsc.subcore_barrier()
    wcps = []
    for q, (off, ln) in enumerate(((0, 160), (160, 160), (320, 160), (480, 152))):
        sl = pl.ds(base_row + off, ln)
        wcps.append(pltpu.async_copy(acc.at[sl], parts_out.at[c].at[sl],
                                     wsems[q]))
    for cp in wcps:
        cp.wait()


_prop_kernel = pl.kernel(
    _prop_body,
    out_type=jax.ShapeDtypeStruct((2, PROP_NP, H), _f32),
    mesh=_mesh,
    compiler_params=_sc_params,
    scratch_types=[
        pltpu.VMEM((P_WCH * P_CH,), _i32),
        pltpu.VMEM((P_WCH * P_CH,), _i32),
        pltpu.VMEM((P_WCH * P_CH,), _f32),
        [pltpu.VMEM((P_CH, H), _f32) for _ in range(NBUF)],
        [pltpu.SemaphoreType.DMA for _ in range(NBUF)],
        [pltpu.SemaphoreType.DMA for _ in range(NBUF)],
        [pltpu.SemaphoreType.DMA for _ in range(3)],
        [pltpu.SemaphoreType.DMA for _ in range(4)],
        pltpu.VMEM_SHARED((PROP_NP, H), _f32),
    ],
)


BR = 1000


def _k1_body(x_ref, w1_ref, dinv_ref, out_ref):
    h = lax.dot_general(x_ref[...], w1_ref[...], (((1,), (0,)), ((), ())),
                        preferred_element_type=_f32)
    out_ref[...] = h * dinv_ref[...]


def _tc_k1(x, W1, dinv2d):
    return pl.pallas_call(
        _k1_body,
        grid=(N // BR,),
        in_specs=[
            pl.BlockSpec((BR, D), lambda i: (i, 0)),
            pl.BlockSpec((D, H), lambda i: (0, 0)),
            pl.BlockSpec((BR, 1), lambda i: (i, 0)),
        ],
        out_specs=pl.BlockSpec((BR, H), lambda i: (i, 0)),
        out_shape=jax.ShapeDtypeStruct((N, H), _f32),
    )(x, W1, dinv2d)


def _k2_body(parts_ref, dinv_ref, b1_ref, z_ref, sums_ref):
    i = pl.program_id(0)
    z = dinv_ref[...] * (parts_ref[0] + parts_ref[1]) + b1_ref[...]
    z_ref[...] = z

    @pl.when(i == 0)
    def _():
        sums_ref[...] = jnp.zeros_like(sums_ref)

    sums_ref[0:1, :] += jnp.sum(z, axis=0, keepdims=True)
    sums_ref[1:2, :] += jnp.sum(z * z, axis=0, keepdims=True)


def _tc_k2(parts, dinv2d, b1):
    return pl.pallas_call(
        _k2_body,
        grid=(N // BR,),
        in_specs=[
            pl.BlockSpec((2, BR, H), lambda i: (0, i, 0)),
            pl.BlockSpec((BR, 1), lambda i: (i, 0)),
            pl.BlockSpec((H,), lambda i: (0,)),
        ],
        out_specs=[
            pl.BlockSpec((BR, H), lambda i: (i, 0)),
            pl.BlockSpec((2, H), lambda i: (0, 0)),
        ],
        out_shape=[
            jax.ShapeDtypeStruct((N, H), _f32),
            jax.ShapeDtypeStruct((2, H), _f32),
        ],
    )(parts, dinv2d, b1)


def _k3_body(z_ref, sums_ref, g1_ref, be1_ref, w2_ref, dinv_ref, out_ref):
    inv_n = 1.0 / N
    m = sums_ref[0:1, :] * inv_n
    var = sums_ref[1:2, :] * inv_n - m * m
    scale = g1_ref[...] * lax.rsqrt(var + 1e-5)
    h = jnp.maximum((z_ref[...] - m) * scale + be1_ref[...], 0.0)
    h2 = lax.dot_general(h, w2_ref[...], (((1,), (0,)), ((), ())),
                         preferred_element_type=_f32)
    out_ref[...] = h2 * dinv_ref[...]


def _tc_k3(z, sums, g1, be1, W2, dinv2d):
    return pl.pallas_call(
        _k3_body,
        grid=(N // BR,),
        in_specs=[
            pl.BlockSpec((BR, H), lambda i: (i, 0)),
            pl.BlockSpec((2, H), lambda i: (0, 0)),
            pl.BlockSpec((H,), lambda i: (0,)),
            pl.BlockSpec((H,), lambda i: (0,)),
            pl.BlockSpec((H, H), lambda i: (0, 0)),
            pl.BlockSpec((BR, 1), lambda i: (i, 0)),
        ],
        out_specs=pl.BlockSpec((BR, H), lambda i: (i, 0)),
        out_shape=jax.ShapeDtypeStruct((N, H), _f32),
    )(z, sums, g1, be1, W2, dinv2d)


def _k4_body(parts_ref, dinv_ref, b2_ref, wc_ref, bc_ref, out_ref):
    z2 = dinv_ref[...] * (parts_ref[0] + parts_ref[1]) + b2_ref[...]
    out = lax.dot_general(z2, wc_ref[...], (((1,), (0,)), ((), ())),
                          preferred_element_type=_f32)
    out_ref[...] = out + bc_ref[...]


def _tc_k4(parts, dinv2d, b2, Wc, bc):
    return pl.pallas_call(
        _k4_body,
        grid=(N // BR,),
        in_specs=[
            pl.BlockSpec((2, BR, H), lambda i: (0, i, 0)),
            pl.BlockSpec((BR, 1), lambda i: (i, 0)),
            pl.BlockSpec((H,), lambda i: (0,)),
            pl.BlockSpec((H, O), lambda i: (0, 0)),
            pl.BlockSpec((O,), lambda i: (0,)),
        ],
        out_specs=pl.BlockSpec((BR, O), lambda i: (i, 0)),
        out_shape=jax.ShapeDtypeStruct((N, O), _f32),
    )(parts, dinv2d, b2, Wc, bc)


def kernel(x, edge_index, edge_weight, W1, b1, g1, be1, W2, b2, g2, be2, Wc, bc):
    src = edge_index[0].astype(_i32)
    dst = edge_index[1].astype(_i32)
    loop = jnp.arange(N, dtype=_i32)
    src_a = jnp.concatenate([src, loop])
    dst_a = jnp.concatenate([dst, loop])
    ew_a = jnp.concatenate([edge_weight.astype(_f32), jnp.ones((N,), _f32)])

    pp = P_TOT - EA
    pad_p = jnp.arange(pp, dtype=_i32) % N
    src_p = jnp.concatenate([src_a, jnp.zeros((pp,), _i32)])
    dst_pc = jnp.concatenate([dst_a, pad_p])
    ew_p = jnp.concatenate([ew_a, jnp.zeros((pp,), _f32)])

    dp = D_TOT - EA
    pad_d = jnp.arange(dp, dtype=_i32) % N
    dst_dc = jnp.concatenate([dst_a, pad_d]).reshape(16, D_NCH, P_CH)
    ew_d = jnp.concatenate([ew_a, jnp.zeros((dp,), _f32)]).reshape(16, D_PER_TILE)

    zdeg = jnp.zeros((ROWS_PER_TILE,), _f32)

    dinv = _deg_kernel(dst_dc, ew_d, zdeg)
    dinv2d = dinv[:N].reshape(N, 1)

    h1p = _tc_k1(x, W1, dinv2d)
    parts1 = _prop_kernel(h1p, src_p, dst_pc, ew_p)
    z, sums = _tc_k2(parts1, dinv2d, b1)
    h2p = _tc_k3(z, sums, g1, be1, W2, dinv2d)
    parts2 = _prop_kernel(h2p, src_p, dst_pc, ew_p)
    return _tc_k4(parts2, dinv2d, b2, Wc, bc)


# speedup vs baseline: 1.2281x; 1.0253x over previous
import functools

import jax
import jax.numpy as jnp
from jax import lax
from jax.experimental import pallas as pl
from jax.experimental.pallas import tpu as pltpu
from jax.experimental.pallas import tpu_sc as plsc

N = 10000
NP = 10240
D = 256
H = 128
O = 70
E = 160000
EA = E + N

P_CH = 128
P_WCH = 30
P_NCH = 42
P_PER_TILE = P_NCH * P_CH
P_TOT = 32 * P_PER_TILE
P_NCHUNKS = P_TOT // P_CH
C0_CH = 58
C1_CH = 26
D_NCH = 86
D_PER_TILE = D_NCH * P_CH
D_TOT = 16 * D_PER_TILE

ROWS_PER_TILE = NP // 16
PROP_ROWS = 632
PROP_NP = 16 * PROP_ROWS

_mesh = plsc.VectorSubcoreMesh(core_axis_name="c", subcore_axis_name="s")
_sc_params = pltpu.CompilerParams(needs_layout_passes=False)

_f32 = jnp.float32
_i32 = jnp.int32


def _rsqrt_newton(x):
    i = lax.bitcast_convert_type(x, _i32)
    i = jnp.full((16,), 0x5F3759DF, _i32) - lax.shift_right_logical(i, 1)
    y = lax.bitcast_convert_type(i, _f32)
    for _ in range(3):
        y = y * (1.5 - 0.5 * x * y * y)
    return y


def _deg_body(dst_c, ew_f, zdeg, dinv_out, dstv, ewv, degl, dinvl, acc):
    c = lax.axis_index("c")
    s = lax.axis_index("s")
    pltpu.sync_copy(dst_c.at[s], dstv)
    pltpu.sync_copy(ew_f.at[s], ewv)
    pltpu.sync_copy(zdeg, acc.at[pl.ds(s * ROWS_PER_TILE, ROWS_PER_TILE)])
    plsc.subcore_barrier()

    def chunk(j, _):
        pltpu.sync_copy(ewv.at[pl.ds(j * P_CH, P_CH)], acc.at[dstv.at[j]],
                        add=True)
        return 0

    lax.fori_loop(0, D_NCH, chunk, 0)
    plsc.subcore_barrier()

    base = (c * 16 + s) * (ROWS_PER_TILE // 2)
    pltpu.sync_copy(acc.at[pl.ds(base, ROWS_PER_TILE // 2)], degl)

    def grp(g, _):
        x = degl[pl.ds(g * 16, 16)]
        dinvl[pl.ds(g * 16, 16)] = _rsqrt_newton(x)
        return 0

    lax.fori_loop(0, ROWS_PER_TILE // 32, grp, 0)
    pltpu.sync_copy(dinvl, dinv_out.at[pl.ds(base, ROWS_PER_TILE // 2)])


_deg_kernel = pl.kernel(
    _deg_body,
    out_type=jax.ShapeDtypeStruct((NP,), _f32),
    mesh=_mesh,
    compiler_params=_sc_params,
    scratch_types=[
        pltpu.VMEM((D_NCH, P_CH), _i32),
        pltpu.VMEM((D_PER_TILE,), _f32),
        pltpu.VMEM((ROWS_PER_TILE // 2,), _f32),
        pltpu.VMEM((ROWS_PER_TILE // 2,), _f32),
        pltpu.VMEM_SHARED((NP,), _f32),
    ],
)


NBUF = 2


def _prop_body(h_t, src_f, dst_c, ew_f, parts_out,
               srcv, dstv, ewv, rows, gsems, ssems, isems, wsems, acc):
    c = lax.axis_index("c")
    s = lax.axis_index("s")
    base_row = s * PROP_ROWS
    zv = jnp.zeros((16,), _f32)

    def zrow(i, _):
        for zc in range(H // 16):
            rows[0][i, pl.ds(zc * 16, 16)] = zv
        return 0

    lax.fori_loop(0, P_CH, zrow, 0)
    zcps = []
    for q, (off, ln) in enumerate(
            ((0, 128), (128, 128), (256, 128), (384, 128), (512, 120))):
        sem = wsems[q] if q < 4 else gsems[0]
        zcps.append(pltpu.async_copy(
            rows[0].at[pl.ds(0, ln)],
            acc.at[pl.ds(base_row + off, ln)], sem))
    for cp in zcps:
        cp.wait()
    plsc.subcore_barrier()

    def do_wave(base_ch, nch):
        ic0 = pltpu.async_copy(src_f.at[pl.ds(base_ch * P_CH, nch * P_CH)],
                               srcv.at[pl.ds(0, nch * P_CH)], isems[0])
        ic1 = pltpu.async_copy(dst_c.at[pl.ds(base_ch * P_CH, nch * P_CH)],
                               dstv.at[pl.ds(0, nch * P_CH)], isems[1])
        ic2 = pltpu.async_copy(ew_f.at[pl.ds(base_ch * P_CH, nch * P_CH)],
                               ewv.at[pl.ds(0, nch * P_CH)], isems[2])
        ic0.wait()
        ic1.wait()
        ic2.wait()

        def group(t, _):
            j0 = t * NBUF
            gets = []
            for b in range(NBUF):
                idx = srcv.at[pl.ds((j0 + b) * P_CH, P_CH)]
                gets.append(pltpu.async_copy(h_t.at[idx], rows[b], gsems[b]))
            puts = []
            for b in range(NBUF):
                gets[b].wait()

                def rbody(r2, _, _b=b):
                    r = r2 * 2
                    for rr in range(2):
                        f = plsc.load_gather(
                            ewv,
                            [jnp.full((16,), (j0 + _b) * P_CH + r + rr,
                                      _i32)])
                        for cc in range(H // 16):
                            sl = pl.ds(cc * 16, 16)
                            rows[_b][r + rr, sl] = rows[_b][r + rr, sl] * f
                    return 0

                lax.fori_loop(0, P_CH // 2, rbody, 0)
                didx = dstv.at[pl.ds((j0 + b) * P_CH, P_CH)]
                puts.append(pltpu.async_copy(rows[b], acc.at[didx],
                                             ssems[b], add=True))
            for cp in puts:
                cp.wait()
            return 0

        lax.fori_loop(0, nch // NBUF, group, 0)

    @pl.when(c == 0)
    def _():
        do_wave(s * C0_CH, 28)
        do_wave(s * C0_CH + 28, 30)

    @pl.when(c == 1)
    def _():
        do_wave(16 * C0_CH + s * C1_CH, 26)

    plsc.subcore_barrier()
    wcps = []
    for q, (off, ln) in enumerate(((0, 160), (160, 160), (320, 160), (480, 152))):
        sl = pl.ds(base_row + off, ln)
        wcps.append(pltpu.async_copy(acc.at[sl], parts_out.at[c].at[sl],
                                     wsems[q]))
    for cp in wcps:
        cp.wait()


_prop_kernel = pl.kernel(
    _prop_body,
    out_type=jax.ShapeDtypeStruct((2, PROP_NP, H), _f32),
    mesh=_mesh,
    compiler_params=_sc_params,
    scratch_types=[
        pltpu.VMEM((P_WCH * P_CH,), _i32),
        pltpu.VMEM((P_WCH * P_CH,), _i32),
        pltpu.VMEM((P_WCH * P_CH,), _f32),
        [pltpu.VMEM((P_CH, H), _f32) for _ in range(NBUF)],
        [pltpu.SemaphoreType.DMA for _ in range(NBUF)],
        [pltpu.SemaphoreType.DMA for _ in range(NBUF)],
        [pltpu.SemaphoreType.DMA for _ in range(3)],
        [pltpu.SemaphoreType.DMA for _ in range(4)],
        pltpu.VMEM_SHARED((PROP_NP, H), _f32),
    ],
)


BR = 1000


def _k1_body(x_ref, w1_ref, dinv_ref, out_ref):
    h = lax.dot_general(x_ref[...], w1_ref[...], (((1,), (0,)), ((), ())),
                        preferred_element_type=_f32)
    out_ref[...] = h * dinv_ref[...]


def _tc_k1(x, W1, dinv2d):
    return pl.pallas_call(
        _k1_body,
        grid=(N // BR,),
        in_specs=[
            pl.BlockSpec((BR, D), lambda i: (i, 0)),
            pl.BlockSpec((D, H), lambda i: (0, 0)),
            pl.BlockSpec((BR, 1), lambda i: (i, 0)),
        ],
        out_specs=pl.BlockSpec((BR, H), lambda i: (i, 0)),
        out_shape=jax.ShapeDtypeStruct((N, H), _f32),
    )(x, W1, dinv2d)


def _k2_body(parts_ref, dinv_ref, b1_ref, z_ref, sums_ref):
    i = pl.program_id(0)
    z = dinv_ref[...] * (parts_ref[0] + parts_ref[1]) + b1_ref[...]
    z_ref[...] = z

    @pl.when(i == 0)
    def _():
        sums_ref[...] = jnp.zeros_like(sums_ref)

    sums_ref[0:1, :] += jnp.sum(z, axis=0, keepdims=True)
    sums_ref[1:2, :] += jnp.sum(z * z, axis=0, keepdims=True)


def _tc_k2(parts, dinv2d, b1):
    return pl.pallas_call(
        _k2_body,
        grid=(N // BR,),
        in_specs=[
            pl.BlockSpec((2, BR, H), lambda i: (0, i, 0)),
            pl.BlockSpec((BR, 1), lambda i: (i, 0)),
            pl.BlockSpec((H,), lambda i: (0,)),
        ],
        out_specs=[
            pl.BlockSpec((BR, H), lambda i: (i, 0)),
            pl.BlockSpec((2, H), lambda i: (0, 0)),
        ],
        out_shape=[
            jax.ShapeDtypeStruct((N, H), _f32),
            jax.ShapeDtypeStruct((2, H), _f32),
        ],
    )(parts, dinv2d, b1)


def _k3_body(z_ref, sums_ref, g1_ref, be1_ref, w2_ref, dinv_ref, out_ref):
    inv_n = 1.0 / N
    m = sums_ref[0:1, :] * inv_n
    var = sums_ref[1:2, :] * inv_n - m * m
    scale = g1_ref[...] * lax.rsqrt(var + 1e-5)
    h = jnp.maximum((z_ref[...] - m) * scale + be1_ref[...], 0.0)
    h2 = lax.dot_general(h, w2_ref[...], (((1,), (0,)), ((), ())),
                         preferred_element_type=_f32)
    out_ref[...] = h2 * dinv_ref[...]


def _tc_k3(z, sums, g1, be1, W2, dinv2d):
    return pl.pallas_call(
        _k3_body,
        grid=(N // BR,),
        in_specs=[
            pl.BlockSpec((BR, H), lambda i: (i, 0)),
            pl.BlockSpec((2, H), lambda i: (0, 0)),
            pl.BlockSpec((H,), lambda i: (0,)),
            pl.BlockSpec((H,), lambda i: (0,)),
            pl.BlockSpec((H, H), lambda i: (0, 0)),
            pl.BlockSpec((BR, 1), lambda i: (i, 0)),
        ],
        out_specs=pl.BlockSpec((BR, H), lambda i: (i, 0)),
        out_shape=jax.ShapeDtypeStruct((N, H), _f32),
    )(z, sums, g1, be1, W2, dinv2d)


def _k4_body(parts_ref, dinv_ref, b2_ref, wc_ref, bc_ref, out_ref):
    z2 = dinv_ref[...] * (parts_ref[0] + parts_ref[1]) + b2_ref[...]
    out = lax.dot_general(z2, wc_ref[...], (((1,), (0,)), ((), ())),
                          preferred_element_type=_f32)
    out_ref[...] = out + bc_ref[...]


def _tc_k4(parts, dinv2d, b2, Wc, bc):
    return pl.pallas_call(
        _k4_body,
        grid=(N // BR,),
        in_specs=[
            pl.BlockSpec((2, BR, H), lambda i: (0, i, 0)),
            pl.BlockSpec((BR, 1), lambda i: (i, 0)),
            pl.BlockSpec((H,), lambda i: (0,)),
            pl.BlockSpec((H, O), lambda i: (0, 0)),
            pl.BlockSpec((O,), lambda i: (0,)),
        ],
        out_specs=pl.BlockSpec((BR, O), lambda i: (i, 0)),
        out_shape=jax.ShapeDtypeStruct((N, O), _f32),
    )(parts, dinv2d, b2, Wc, bc)


def kernel(x, edge_index, edge_weight, W1, b1, g1, be1, W2, b2, g2, be2, Wc, bc):
    src = edge_index[0].astype(_i32)
    dst = edge_index[1].astype(_i32)
    loop = jnp.arange(N, dtype=_i32)
    src_a = jnp.concatenate([src, loop])
    dst_a = jnp.concatenate([dst, loop])
    ew_a = jnp.concatenate([edge_weight.astype(_f32), jnp.ones((N,), _f32)])

    pp = P_TOT - EA
    pad_p = jnp.arange(pp, dtype=_i32) % N
    src_p = jnp.concatenate([src_a, jnp.zeros((pp,), _i32)])
    dst_pc = jnp.concatenate([dst_a, pad_p])
    ew_p = jnp.concatenate([ew_a, jnp.zeros((pp,), _f32)])

    dp = D_TOT - EA
    pad_d = jnp.arange(dp, dtype=_i32) % N
    dst_dc = jnp.concatenate([dst_a, pad_d]).reshape(16, D_NCH, P_CH)
    ew_d = jnp.concatenate([ew_a, jnp.zeros((dp,), _f32)]).reshape(16, D_PER_TILE)

    zdeg = jnp.zeros((ROWS_PER_TILE,), _f32)

    dinv = _deg_kernel(dst_dc, ew_d, zdeg)
    dinv2d = dinv[:N].reshape(N, 1)

    h1p = _tc_k1(x, W1, dinv2d)
    parts1 = _prop_kernel(h1p, src_p, dst_pc, ew_p)
    z, sums = _tc_k2(parts1, dinv2d, b1)
    h2p = _tc_k3(z, sums, g1, be1, W2, dinv2d)
    parts2 = _prop_kernel(h2p, src_p, dst_pc, ew_p)
    return _tc_k4(parts2, dinv2d, b2, Wc, bc)
